# baseline (speedup 1.0000x reference)
import functools

import jax
import jax.numpy as jnp
from jax import lax
from jax.experimental import pallas as pl
from jax.experimental.pallas import tpu as pltpu
from jax.experimental.pallas import tpu_sc as plsc

N_NODES = 10000
N_EDGES = 320000
NC = 2
NS = 16
EB = 128
NBT = 80
CH = 40
E_PAD = NC * NS * NBT * EB
NP = 10016
RT = 624
TAIL = N_NODES - NS * RT
ZTAIL = NP - NS * RT

_mesh = plsc.VectorSubcoreMesh(core_axis_name="c", subcore_axis_name="s")


def _zero_stripe(z_hbm, sh, s):
    pltpu.sync_copy(z_hbm.at[pl.ds(0, RT)], sh.at[pl.ds(s * RT, RT)])

    @pl.when(s == NS - 1)
    def _():
        pltpu.sync_copy(z_hbm.at[pl.ds(0, ZTAIL)],
                        sh.at[pl.ds(NS * RT, ZTAIL)])


def _write_stripe(sh, out_hbm, c, s):
    pltpu.sync_copy(sh.at[pl.ds(s * RT, RT)],
                    out_hbm.at[c, pl.ds(s * RT, RT), :])

    @pl.when(s == NS - 1)
    def _():
        pltpu.sync_copy(sh.at[pl.ds(NS * RT, TAIL)],
                        out_hbm.at[c, pl.ds(NS * RT, TAIL), :])


def _edge_pipeline(tab_hbm, src_hbm, dst2_hbm, agg_sh, sidx, didx,
                   rows0, rows1, g0, g1, s0, s1, tile_b0):
    for h in range(NBT // CH):
        cb = tile_b0 + h * CH
        pltpu.sync_copy(src_hbm.at[pl.ds(cb * EB, CH * EB)], sidx)
        pltpu.sync_copy(dst2_hbm.at[pl.ds(cb, CH)], didx)

        def pair(p, _):
            j0 = 2 * p
            j1 = 2 * p + 1
            gd0 = pltpu.async_copy(
                tab_hbm.at[sidx.at[pl.ds(j0 * EB, EB)]], rows0, g0)
            gd1 = pltpu.async_copy(
                tab_hbm.at[sidx.at[pl.ds(j1 * EB, EB)]], rows1, g1)
            gd0.wait()
            sd0 = pltpu.async_copy(rows0, agg_sh.at[didx.at[j0]], s0,
                                   add=True)
            gd1.wait()
            sd1 = pltpu.async_copy(rows1, agg_sh.at[didx.at[j1]], s1,
                                   add=True)
            sd0.wait()
            sd1.wait()
            return 0

        lax.fori_loop(0, CH // 2, pair, 0)


def _make_segsum_edge_split():
    scratch = [
        pltpu.VMEM((CH * EB,), jnp.int32),
        pltpu.VMEM((CH, EB), jnp.int32),
        pltpu.VMEM((EB, 128), jnp.float32),
        pltpu.VMEM((EB, 128), jnp.float32),
        pltpu.VMEM_SHARED((NP, 128), jnp.float32),
    ] + [pltpu.SemaphoreType.DMA] * 4

    def body(tab_hbm, src_hbm, dst2_hbm, z128_hbm, out_hbm,
             sidx, didx, rows0, rows1, agg_sh, g0, g1, s0, s1):
        c = lax.axis_index("c")
        s = lax.axis_index("s")
        _zero_stripe(z128_hbm, agg_sh, s)
        plsc.subcore_barrier()
        tile_b0 = (c * NS + s) * NBT
        _edge_pipeline(tab_hbm, src_hbm, dst2_hbm, agg_sh, sidx, didx,
                       rows0, rows1, g0, g1, s0, s1, tile_b0)
        plsc.subcore_barrier()
        _write_stripe(agg_sh, out_hbm, c, s)

    return pl.kernel(
        body, out_type=jax.ShapeDtypeStruct((NC, N_NODES, 128), jnp.float32),
        mesh=_mesh, scratch_types=scratch)


def _make_segsum_channel_split():
    nbt_all = NBT * NC

    scratch = [
        pltpu.VMEM((CH * EB,), jnp.int32),
        pltpu.VMEM((CH, EB), jnp.int32),
        pltpu.VMEM((EB, 128), jnp.float32),
        pltpu.VMEM((EB, 128), jnp.float32),
        pltpu.VMEM_SHARED((NP, 128), jnp.float32),
    ] + [pltpu.SemaphoreType.DMA] * 4

    def body(t0_hbm, t1_hbm, src_hbm, dst2_hbm, z128_hbm, out_hbm,
             sidx, didx, rows0, rows1, agg_sh, g0, g1, s0, s1):
        c = lax.axis_index("c")
        s = lax.axis_index("s")
        _zero_stripe(z128_hbm, agg_sh, s)
        plsc.subcore_barrier()
        tile_b0 = s * nbt_all

        @pl.when(c == 0)
        def _():
            for h2 in range(NC):
                _edge_pipeline(t0_hbm, src_hbm, dst2_hbm, agg_sh, sidx,
                               didx, rows0, rows1, g0, g1, s0, s1,
                               tile_b0 + h2 * NBT)

        @pl.when(c == 1)
        def _():
            for h2 in range(NC):
                _edge_pipeline(t1_hbm, src_hbm, dst2_hbm, agg_sh, sidx,
                               didx, rows0, rows1, g0, g1, s0, s1,
                               tile_b0 + h2 * NBT)

        plsc.subcore_barrier()
        _write_stripe(agg_sh, out_hbm, c, s)

    return pl.kernel(
        body, out_type=jax.ShapeDtypeStruct((NC, N_NODES, 128), jnp.float32),
        mesh=_mesh, scratch_types=scratch)


def _make_deg():
    scratch = [
        pltpu.VMEM((CH, EB), jnp.int32),
        pltpu.VMEM((EB, 128), jnp.float32),
        pltpu.VMEM_SHARED((NP, 128), jnp.float32),
    ] + [pltpu.SemaphoreType.DMA] * 2

    def body(dst2_hbm, z128_hbm, ones_hbm, out_hbm, didx, ones_v, deg_sh,
             s0, s1):
        c = lax.axis_index("c")
        s = lax.axis_index("s")
        _zero_stripe(z128_hbm, deg_sh, s)
        pltpu.sync_copy(ones_hbm, ones_v)
        plsc.subcore_barrier()
        tile_b0 = (c * NS + s) * NBT
        for h in range(NBT // CH):
            cb = tile_b0 + h * CH
            pltpu.sync_copy(dst2_hbm.at[pl.ds(cb, CH)], didx)

            def pair(p, _):
                sd0 = pltpu.async_copy(
                    ones_v, deg_sh.at[didx.at[2 * p]], s0, add=True)
                sd1 = pltpu.async_copy(
                    ones_v, deg_sh.at[didx.at[2 * p + 1]], s1, add=True)
                sd0.wait()
                sd1.wait()
                return 0

            lax.fori_loop(0, CH // 2, pair, 0)
        plsc.subcore_barrier()
        _write_stripe(deg_sh, out_hbm, c, s)

    return pl.kernel(
        body, out_type=jax.ShapeDtypeStruct((NC, N_NODES, 128), jnp.float32),
        mesh=_mesh, scratch_types=scratch)


_segsum_edges = _make_segsum_edge_split()
_segsum_chans = _make_segsum_channel_split()
_deg_counts = _make_deg()


BN = 2000


def _recip_deg(degp_ref):
    deg = degp_ref[0, :, 0:1] + degp_ref[1, :, 0:1]
    return 1.0 / jnp.maximum(deg, 1.0)


def _dot(a, b):
    return jnp.dot(a, b, preferred_element_type=jnp.float32)


def _layer0_body(pa_ref, degp_ref, x_ref, wl_ref, bl_ref, wr_ref,
                 outa_ref, outb_ref):
    mean = (pa_ref[0] + pa_ref[1]) * _recip_deg(degp_ref)
    h = _dot(mean, wl_ref[...]) + bl_ref[...] + _dot(x_ref[...], wr_ref[...])
    h = jnp.maximum(h, 0.0)
    outa_ref[...] = h[:, :128]
    outb_ref[...] = h[:, 128:]


def _layer1_body(agg_ref, degp_ref, h1a_ref, h1b_ref, wla_ref, wlb_ref,
                 bl_ref, wra_ref, wrb_ref, wl2_ref,
                 outa_ref, outb_ref, outm_ref):
    recip = _recip_deg(degp_ref)
    h = (_dot(agg_ref[0] * recip, wla_ref[...])
         + _dot(agg_ref[1] * recip, wlb_ref[...])
         + bl_ref[...]
         + _dot(h1a_ref[...], wra_ref[...])
         + _dot(h1b_ref[...], wrb_ref[...]))
    h = jnp.maximum(h, 0.0)
    outa_ref[...] = h[:, :128]
    outb_ref[...] = h[:, 128:]
    outm_ref[...] = _dot(h, wl2_ref[...])


def _layer2_body(pm_ref, degp_ref, h2a_ref, h2b_ref, wra_ref, wrb_ref,
                 bl_ref, out_ref):
    mean_wl = (pm_ref[0] + pm_ref[1]) * _recip_deg(degp_ref)
    out_ref[...] = (mean_wl + bl_ref[...]
                    + _dot(h2a_ref[...], wra_ref[...])
                    + _dot(h2b_ref[...], wrb_ref[...]))


def _node_spec(ch):
    return pl.BlockSpec((NC, BN, ch), lambda i: (0, i, 0))


def _row_spec(ch):
    return pl.BlockSpec((BN, ch), lambda i: (i, 0))


def _full_spec(shape):
    n = len(shape)
    return pl.BlockSpec(shape, lambda i: (0,) * n)


_GRID = (N_NODES // BN,)


def _layer0(pa, degp, x, wl, bl, wr):
    return pl.pallas_call(
        _layer0_body,
        grid=_GRID,
        in_specs=[_node_spec(128), _node_spec(128), _row_spec(128),
                  _full_spec(wl.shape), _full_spec(bl.shape),
                  _full_spec(wr.shape)],
        out_specs=[_row_spec(128), _row_spec(128)],
        out_shape=[jax.ShapeDtypeStruct((N_NODES, 128), jnp.float32)] * 2,
    )(pa, degp, x, wl, bl, wr)


def _layer1(agg, degp, h1a, h1b, wla, wlb, bl, wra, wrb, wl2):
    return pl.pallas_call(
        _layer1_body,
        grid=_GRID,
        in_specs=[_node_spec(128), _node_spec(128), _row_spec(128),
                  _row_spec(128), _full_spec(wla.shape),
                  _full_spec(wlb.shape), _full_spec(bl.shape),
                  _full_spec(wra.shape), _full_spec(wrb.shape),
                  _full_spec(wl2.shape)],
        out_specs=[_row_spec(128), _row_spec(128), _row_spec(128)],
        out_shape=[jax.ShapeDtypeStruct((N_NODES, 128), jnp.float32)] * 3,
    )(agg, degp, h1a, h1b, wla, wlb, bl, wra, wrb, wl2)


def _layer2(pm, degp, h2a, h2b, wra, wrb, bl):
    return pl.pallas_call(
        _layer2_body,
        grid=_GRID,
        in_specs=[_node_spec(128), _node_spec(128), _row_spec(128),
                  _row_spec(128), _full_spec(wra.shape),
                  _full_spec(wrb.shape), _full_spec(bl.shape)],
        out_specs=pl.BlockSpec((BN, 128), lambda i: (i, 0)),
        out_shape=jax.ShapeDtypeStruct((N_NODES, 128), jnp.float32),
    )(pm, degp, h2a, h2b, wra, wrb, bl)


@jax.jit
def kernel(x, adj_t, Wl0, bl0, Wr0, Wl1, bl1, Wr1, Wl2, bl2, Wr2):
    src = adj_t[0].astype(jnp.int32)
    dst = adj_t[1].astype(jnp.int32)
    npad = E_PAD - N_EDGES
    src_p = jnp.concatenate([src, jnp.zeros((npad,), jnp.int32)])
    dst2 = jnp.concatenate(
        [dst, jnp.full((npad,), N_NODES, jnp.int32)]).reshape(-1, EB)
    z128 = jnp.zeros((RT, 128), jnp.float32)
    ones128 = jnp.ones((EB, 128), jnp.float32)

    degp = _deg_counts(dst2, z128, ones128)
    pa = _segsum_edges(x, src_p, dst2, z128)
    h1a, h1b = _layer0(pa, degp, x, Wl0, bl0.reshape(1, -1), Wr0)

    agg1 = _segsum_chans(h1a, h1b, src_p, dst2, z128)
    h2a, h2b, m = _layer1(agg1, degp, h1a, h1b, Wl1[:128], Wl1[128:],
                          bl1.reshape(1, -1), Wr1[:128], Wr1[128:], Wl2)

    pm = _segsum_edges(m, src_p, dst2, z128)
    out = _layer2(pm, degp, h2a, h2b, Wr2[:128], Wr2[128:],
                  bl2.reshape(1, -1))
    return out


# speedup vs baseline: 2.6703x; 2.6703x over previous
import functools

import jax
import jax.numpy as jnp
from jax import lax
from jax.experimental import pallas as pl
from jax.experimental.pallas import tpu as pltpu
from jax.experimental.pallas import tpu_sc as plsc

N_NODES = 10000
N_EDGES = 320000
NC = 2
NS = 16
EB = 128
NBT = 80
CH = 40
E_PAD = NC * NS * NBT * EB
NP = 10016
RT = 624
TAIL = N_NODES - NS * RT
ZTAIL = NP - NS * RT

_mesh = plsc.VectorSubcoreMesh(core_axis_name="c", subcore_axis_name="s")


def _zero_stripe(z_hbm, sh, s):
    pltpu.sync_copy(z_hbm.at[pl.ds(0, RT)], sh.at[pl.ds(s * RT, RT)])

    @pl.when(s == NS - 1)
    def _():
        pltpu.sync_copy(z_hbm.at[pl.ds(0, ZTAIL)],
                        sh.at[pl.ds(NS * RT, ZTAIL)])


def _write_stripe(sh, out_hbm, c, s):
    pltpu.sync_copy(sh.at[pl.ds(s * RT, RT)],
                    out_hbm.at[c, pl.ds(s * RT, RT), :])

    @pl.when(s == NS - 1)
    def _():
        pltpu.sync_copy(sh.at[pl.ds(NS * RT, TAIL)],
                        out_hbm.at[c, pl.ds(NS * RT, TAIL), :])


def _edge_pipeline(tab_hbm, src_hbm, dst2_hbm, agg_sh, sidx, didx,
                   rows0, rows1, g0, g1, s0, s1, tile_b0):
    for h in range(NBT // CH):
        cb = tile_b0 + h * CH
        pltpu.sync_copy(src_hbm.at[pl.ds(cb * EB, CH * EB)], sidx)
        pltpu.sync_copy(dst2_hbm.at[pl.ds(cb, CH)], didx)

        def pair(p, _):
            j0 = 2 * p
            j1 = 2 * p + 1
            gd0 = pltpu.async_copy(
                tab_hbm.at[sidx.at[pl.ds(j0 * EB, EB)]], rows0, g0)
            gd1 = pltpu.async_copy(
                tab_hbm.at[sidx.at[pl.ds(j1 * EB, EB)]], rows1, g1)
            gd0.wait()
            sd0 = pltpu.async_copy(rows0, agg_sh.at[didx.at[j0]], s0,
                                   add=True)
            gd1.wait()
            sd1 = pltpu.async_copy(rows1, agg_sh.at[didx.at[j1]], s1,
                                   add=True)
            sd0.wait()
            sd1.wait()
            return 0

        lax.fori_loop(0, CH // 2, pair, 0)


def _make_segsum_edge_split():
    scratch = [
        pltpu.VMEM((CH * EB,), jnp.int32),
        pltpu.VMEM((CH, EB), jnp.int32),
        pltpu.VMEM((EB, 128), jnp.float32),
        pltpu.VMEM((EB, 128), jnp.float32),
        pltpu.VMEM_SHARED((NP, 128), jnp.float32),
    ] + [pltpu.SemaphoreType.DMA] * 4

    def body(tab_hbm, src_hbm, dst2_hbm, z128_hbm, out_hbm,
             sidx, didx, rows0, rows1, agg_sh, g0, g1, s0, s1):
        c = lax.axis_index("c")
        s = lax.axis_index("s")
        _zero_stripe(z128_hbm, agg_sh, s)
        plsc.subcore_barrier()
        tile_b0 = (c * NS + s) * NBT
        _edge_pipeline(tab_hbm, src_hbm, dst2_hbm, agg_sh, sidx, didx,
                       rows0, rows1, g0, g1, s0, s1, tile_b0)
        plsc.subcore_barrier()
        _write_stripe(agg_sh, out_hbm, c, s)

    return pl.kernel(
        body, out_type=jax.ShapeDtypeStruct((NC, N_NODES, 128), jnp.float32),
        mesh=_mesh, scratch_types=scratch)


def _make_segsum_channel_split():
    nbt_all = NBT * NC

    scratch = [
        pltpu.VMEM((CH * EB,), jnp.int32),
        pltpu.VMEM((CH, EB), jnp.int32),
        pltpu.VMEM((EB, 128), jnp.float32),
        pltpu.VMEM((EB, 128), jnp.float32),
        pltpu.VMEM_SHARED((NP, 128), jnp.float32),
    ] + [pltpu.SemaphoreType.DMA] * 4

    def body(t0_hbm, t1_hbm, src_hbm, dst2_hbm, z128_hbm, out_hbm,
             sidx, didx, rows0, rows1, agg_sh, g0, g1, s0, s1):
        c = lax.axis_index("c")
        s = lax.axis_index("s")
        _zero_stripe(z128_hbm, agg_sh, s)
        plsc.subcore_barrier()
        tile_b0 = s * nbt_all

        @pl.when(c == 0)
        def _():
            for h2 in range(NC):
                _edge_pipeline(t0_hbm, src_hbm, dst2_hbm, agg_sh, sidx,
                               didx, rows0, rows1, g0, g1, s0, s1,
                               tile_b0 + h2 * NBT)

        @pl.when(c == 1)
        def _():
            for h2 in range(NC):
                _edge_pipeline(t1_hbm, src_hbm, dst2_hbm, agg_sh, sidx,
                               didx, rows0, rows1, g0, g1, s0, s1,
                               tile_b0 + h2 * NBT)

        plsc.subcore_barrier()
        _write_stripe(agg_sh, out_hbm, c, s)

    return pl.kernel(
        body, out_type=jax.ShapeDtypeStruct((NC, N_NODES, 128), jnp.float32),
        mesh=_mesh, scratch_types=scratch)


def _make_deg():
    scratch = [
        pltpu.VMEM((CH, EB), jnp.int32),
        pltpu.VMEM((EB, 128), jnp.float32),
        pltpu.VMEM_SHARED((NP, 128), jnp.float32),
    ] + [pltpu.SemaphoreType.DMA] * 2

    def body(dst2_hbm, z128_hbm, ones_hbm, out_hbm, didx, ones_v, deg_sh,
             s0, s1):
        c = lax.axis_index("c")
        s = lax.axis_index("s")
        _zero_stripe(z128_hbm, deg_sh, s)
        pltpu.sync_copy(ones_hbm, ones_v)
        plsc.subcore_barrier()
        tile_b0 = (c * NS + s) * NBT
        for h in range(NBT // CH):
            cb = tile_b0 + h * CH
            pltpu.sync_copy(dst2_hbm.at[pl.ds(cb, CH)], didx)

            def pair(p, _):
                sd0 = pltpu.async_copy(
                    ones_v, deg_sh.at[didx.at[2 * p]], s0, add=True)
                sd1 = pltpu.async_copy(
                    ones_v, deg_sh.at[didx.at[2 * p + 1]], s1, add=True)
                sd0.wait()
                sd1.wait()
                return 0

            lax.fori_loop(0, CH // 2, pair, 0)
        plsc.subcore_barrier()
        _write_stripe(deg_sh, out_hbm, c, s)

    return pl.kernel(
        body, out_type=jax.ShapeDtypeStruct((NC, N_NODES, 128), jnp.float32),
        mesh=_mesh, scratch_types=scratch)


_segsum_edges = _make_segsum_edge_split()
_segsum_chans = _make_segsum_channel_split()
_deg_counts = _make_deg()


BN = 2000


def _recip_deg(degp_ref):
    deg = degp_ref[0, :, 0:1] + degp_ref[1, :, 0:1]
    return 1.0 / jnp.maximum(deg, 1.0)


def _dot(a, b):
    return jnp.dot(a, b, preferred_element_type=jnp.float32)


def _layer0_body(pa_ref, degp_ref, x_ref, wl_ref, bl_ref, wr_ref,
                 outa_ref, outb_ref):
    mean = (pa_ref[0] + pa_ref[1]) * _recip_deg(degp_ref)
    h = _dot(mean, wl_ref[...]) + bl_ref[...] + _dot(x_ref[...], wr_ref[...])
    h = jnp.maximum(h, 0.0)
    outa_ref[...] = h[:, :128]
    outb_ref[...] = h[:, 128:]


def _layer1_body(agg_ref, degp_ref, h1a_ref, h1b_ref, wla_ref, wlb_ref,
                 bl_ref, wra_ref, wrb_ref, wl2_ref,
                 outa_ref, outb_ref, outm_ref):
    recip = _recip_deg(degp_ref)
    h = (_dot(agg_ref[0] * recip, wla_ref[...])
         + _dot(agg_ref[1] * recip, wlb_ref[...])
         + bl_ref[...]
         + _dot(h1a_ref[...], wra_ref[...])
         + _dot(h1b_ref[...], wrb_ref[...]))
    h = jnp.maximum(h, 0.0)
    outa_ref[...] = h[:, :128]
    outb_ref[...] = h[:, 128:]
    outm_ref[...] = _dot(h, wl2_ref[...])


def _layer2_body(pm_ref, degp_ref, h2a_ref, h2b_ref, wra_ref, wrb_ref,
                 bl_ref, out_ref):
    mean_wl = (pm_ref[0] + pm_ref[1]) * _recip_deg(degp_ref)
    out_ref[...] = (mean_wl + bl_ref[...]
                    + _dot(h2a_ref[...], wra_ref[...])
                    + _dot(h2b_ref[...], wrb_ref[...]))


def _node_spec(ch):
    return pl.BlockSpec((NC, BN, ch), lambda i: (0, i, 0))


def _row_spec(ch):
    return pl.BlockSpec((BN, ch), lambda i: (i, 0))


def _full_spec(shape):
    n = len(shape)
    return pl.BlockSpec(shape, lambda i: (0,) * n)


_GRID = (N_NODES // BN,)


def _layer0(pa, degp, x, wl, bl, wr):
    return pl.pallas_call(
        _layer0_body,
        grid=_GRID,
        in_specs=[_node_spec(128), _node_spec(128), _row_spec(128),
                  _full_spec(wl.shape), _full_spec(bl.shape),
                  _full_spec(wr.shape)],
        out_specs=[_row_spec(128), _row_spec(128)],
        out_shape=[jax.ShapeDtypeStruct((N_NODES, 128), jnp.float32)] * 2,
    )(pa, degp, x, wl, bl, wr)


def _layer1(agg, degp, h1a, h1b, wla, wlb, bl, wra, wrb, wl2):
    return pl.pallas_call(
        _layer1_body,
        grid=_GRID,
        in_specs=[_node_spec(128), _node_spec(128), _row_spec(128),
                  _row_spec(128), _full_spec(wla.shape),
                  _full_spec(wlb.shape), _full_spec(bl.shape),
                  _full_spec(wra.shape), _full_spec(wrb.shape),
                  _full_spec(wl2.shape)],
        out_specs=[_row_spec(128), _row_spec(128), _row_spec(128)],
        out_shape=[jax.ShapeDtypeStruct((N_NODES, 128), jnp.float32)] * 3,
    )(agg, degp, h1a, h1b, wla, wlb, bl, wra, wrb, wl2)


def _layer2(pm, degp, h2a, h2b, wra, wrb, bl):
    return pl.pallas_call(
        _layer2_body,
        grid=_GRID,
        in_specs=[_node_spec(128), _node_spec(128), _row_spec(128),
                  _row_spec(128), _full_spec(wra.shape),
                  _full_spec(wrb.shape), _full_spec(bl.shape)],
        out_specs=pl.BlockSpec((BN, 128), lambda i: (i, 0)),
        out_shape=jax.ShapeDtypeStruct((N_NODES, 128), jnp.float32),
    )(pm, degp, h2a, h2b, wra, wrb, bl)


@jax.jit
def kernel(x, adj_t, Wl0, bl0, Wr0, Wl1, bl1, Wr1, Wl2, bl2, Wr2):
    src = adj_t[0].astype(jnp.int32)
    dst = adj_t[1].astype(jnp.int32)
    npad = E_PAD - N_EDGES
    pad_ar = jnp.arange(npad, dtype=jnp.int32)
    src_p = jnp.concatenate([src, pad_ar % N_NODES])
    dst2 = jnp.concatenate(
        [dst, N_NODES + (pad_ar % (NP - N_NODES))]).reshape(-1, EB)
    z128 = jnp.zeros((RT, 128), jnp.float32)
    ones128 = jnp.ones((EB, 128), jnp.float32)

    degp = _deg_counts(dst2, z128, ones128)
    pa = _segsum_edges(x, src_p, dst2, z128)
    h1a, h1b = _layer0(pa, degp, x, Wl0, bl0.reshape(1, -1), Wr0)

    agg1 = _segsum_chans(h1a, h1b, src_p, dst2, z128)
    h2a, h2b, m = _layer1(agg1, degp, h1a, h1b, Wl1[:128], Wl1[128:],
                          bl1.reshape(1, -1), Wr1[:128], Wr1[128:], Wl2)

    pm = _segsum_edges(m, src_p, dst2, z128)
    out = _layer2(pm, degp, h2a, h2b, Wr2[:128], Wr2[128:],
                  bl2.reshape(1, -1))
    return out


# speedup vs baseline: 2.7082x; 1.0142x over previous
import functools

import jax
import jax.numpy as jnp
from jax import lax
from jax.experimental import pallas as pl
from jax.experimental.pallas import tpu as pltpu
from jax.experimental.pallas import tpu_sc as plsc

N_NODES = 10000
N_EDGES = 320000
NC = 2
NS = 16
EB = 128
NBT = 80
CH = 40
E_PAD = NC * NS * NBT * EB
NP = 10016
RT = 624
TAIL = N_NODES - NS * RT
ZTAIL = NP - NS * RT

_mesh = plsc.VectorSubcoreMesh(core_axis_name="c", subcore_axis_name="s")


def _zero_stripe(z_hbm, sh, s):
    pltpu.sync_copy(z_hbm.at[pl.ds(0, RT)], sh.at[pl.ds(s * RT, RT)])

    @pl.when(s == NS - 1)
    def _():
        pltpu.sync_copy(z_hbm.at[pl.ds(0, ZTAIL)],
                        sh.at[pl.ds(NS * RT, ZTAIL)])


def _write_stripe(sh, out_hbm, c, s):
    pltpu.sync_copy(sh.at[pl.ds(s * RT, RT)],
                    out_hbm.at[c, pl.ds(s * RT, RT), :])

    @pl.when(s == NS - 1)
    def _():
        pltpu.sync_copy(sh.at[pl.ds(NS * RT, TAIL)],
                        out_hbm.at[c, pl.ds(NS * RT, TAIL), :])


def _edge_pipeline(tab_hbm, src_hbm, dst2_hbm, agg_sh, sidx, didx,
                   rows0, rows1, g0, g1, s0, s1, tile_b0):
    def _gather(j, rows, sem):
        return tab_hbm.at[sidx.at[pl.ds(j * EB, EB)]], rows, sem

    for h in range(NBT // CH):
        cb = tile_b0 + h * CH
        pltpu.sync_copy(src_hbm.at[pl.ds(cb * EB, CH * EB)], sidx)
        pltpu.sync_copy(dst2_hbm.at[pl.ds(cb, CH)], didx)

        pltpu.async_copy(*_gather(0, rows0, g0))
        pltpu.async_copy(*_gather(1, rows1, g1))

        def pair(p, _):
            j0 = 2 * p
            j1 = 2 * p + 1
            pltpu.make_async_copy(*_gather(j0, rows0, g0)).wait()
            sd0 = pltpu.async_copy(rows0, agg_sh.at[didx.at[j0]], s0,
                                   add=True)
            pltpu.make_async_copy(*_gather(j1, rows1, g1)).wait()
            sd1 = pltpu.async_copy(rows1, agg_sh.at[didx.at[j1]], s1,
                                   add=True)
            sd0.wait()
            pltpu.async_copy(*_gather(j0 + 2, rows0, g0))
            sd1.wait()
            pltpu.async_copy(*_gather(j1 + 2, rows1, g1))
            return 0

        lax.fori_loop(0, CH // 2 - 1, pair, 0)

        jl = CH - 2
        pltpu.make_async_copy(*_gather(jl, rows0, g0)).wait()
        sd0 = pltpu.async_copy(rows0, agg_sh.at[didx.at[jl]], s0, add=True)
        pltpu.make_async_copy(*_gather(jl + 1, rows1, g1)).wait()
        sd1 = pltpu.async_copy(rows1, agg_sh.at[didx.at[jl + 1]], s1,
                               add=True)
        sd0.wait()
        sd1.wait()


def _make_segsum_edge_split():
    scratch = [
        pltpu.VMEM((CH * EB,), jnp.int32),
        pltpu.VMEM((CH, EB), jnp.int32),
        pltpu.VMEM((EB, 128), jnp.float32),
        pltpu.VMEM((EB, 128), jnp.float32),
        pltpu.VMEM_SHARED((NP, 128), jnp.float32),
    ] + [pltpu.SemaphoreType.DMA] * 4

    def body(tab_hbm, src_hbm, dst2_hbm, z128_hbm, out_hbm,
             sidx, didx, rows0, rows1, agg_sh, g0, g1, s0, s1):
        c = lax.axis_index("c")
        s = lax.axis_index("s")
        _zero_stripe(z128_hbm, agg_sh, s)
        plsc.subcore_barrier()
        tile_b0 = (c * NS + s) * NBT
        _edge_pipeline(tab_hbm, src_hbm, dst2_hbm, agg_sh, sidx, didx,
                       rows0, rows1, g0, g1, s0, s1, tile_b0)
        plsc.subcore_barrier()
        _write_stripe(agg_sh, out_hbm, c, s)

    return pl.kernel(
        body, out_type=jax.ShapeDtypeStruct((NC, N_NODES, 128), jnp.float32),
        mesh=_mesh, scratch_types=scratch)


def _make_segsum_channel_split():
    nbt_all = NBT * NC

    scratch = [
        pltpu.VMEM((CH * EB,), jnp.int32),
        pltpu.VMEM((CH, EB), jnp.int32),
        pltpu.VMEM((EB, 128), jnp.float32),
        pltpu.VMEM((EB, 128), jnp.float32),
        pltpu.VMEM_SHARED((NP, 128), jnp.float32),
    ] + [pltpu.SemaphoreType.DMA] * 4

    def body(t0_hbm, t1_hbm, src_hbm, dst2_hbm, z128_hbm, out_hbm,
             sidx, didx, rows0, rows1, agg_sh, g0, g1, s0, s1):
        c = lax.axis_index("c")
        s = lax.axis_index("s")
        _zero_stripe(z128_hbm, agg_sh, s)
        plsc.subcore_barrier()
        tile_b0 = s * nbt_all

        @pl.when(c == 0)
        def _():
            for h2 in range(NC):
                _edge_pipeline(t0_hbm, src_hbm, dst2_hbm, agg_sh, sidx,
                               didx, rows0, rows1, g0, g1, s0, s1,
                               tile_b0 + h2 * NBT)

        @pl.when(c == 1)
        def _():
            for h2 in range(NC):
                _edge_pipeline(t1_hbm, src_hbm, dst2_hbm, agg_sh, sidx,
                               didx, rows0, rows1, g0, g1, s0, s1,
                               tile_b0 + h2 * NBT)

        plsc.subcore_barrier()
        _write_stripe(agg_sh, out_hbm, c, s)

    return pl.kernel(
        body, out_type=jax.ShapeDtypeStruct((NC, N_NODES, 128), jnp.float32),
        mesh=_mesh, scratch_types=scratch)


def _make_deg():
    scratch = [
        pltpu.VMEM((CH, EB), jnp.int32),
        pltpu.VMEM((EB, 128), jnp.float32),
        pltpu.VMEM_SHARED((NP, 128), jnp.float32),
    ] + [pltpu.SemaphoreType.DMA] * 2

    def body(dst2_hbm, z128_hbm, ones_hbm, out_hbm, didx, ones_v, deg_sh,
             s0, s1):
        c = lax.axis_index("c")
        s = lax.axis_index("s")
        _zero_stripe(z128_hbm, deg_sh, s)
        pltpu.sync_copy(ones_hbm, ones_v)
        plsc.subcore_barrier()
        tile_b0 = (c * NS + s) * NBT
        for h in range(NBT // CH):
            cb = tile_b0 + h * CH
            pltpu.sync_copy(dst2_hbm.at[pl.ds(cb, CH)], didx)

            def pair(p, _):
                sd0 = pltpu.async_copy(
                    ones_v, deg_sh.at[didx.at[2 * p]], s0, add=True)
                sd1 = pltpu.async_copy(
                    ones_v, deg_sh.at[didx.at[2 * p + 1]], s1, add=True)
                sd0.wait()
                sd1.wait()
                return 0

            lax.fori_loop(0, CH // 2, pair, 0)
        plsc.subcore_barrier()
        _write_stripe(deg_sh, out_hbm, c, s)

    return pl.kernel(
        body, out_type=jax.ShapeDtypeStruct((NC, N_NODES, 128), jnp.float32),
        mesh=_mesh, scratch_types=scratch)


_segsum_edges = _make_segsum_edge_split()
_segsum_chans = _make_segsum_channel_split()
_deg_counts = _make_deg()


BN = 2000


def _recip_deg(degp_ref):
    deg = degp_ref[0, :, 0:1] + degp_ref[1, :, 0:1]
    return 1.0 / jnp.maximum(deg, 1.0)


def _dot(a, b):
    return jnp.dot(a, b, preferred_element_type=jnp.float32)


def _layer0_body(pa_ref, degp_ref, x_ref, wl_ref, bl_ref, wr_ref,
                 outa_ref, outb_ref):
    mean = (pa_ref[0] + pa_ref[1]) * _recip_deg(degp_ref)
    h = _dot(mean, wl_ref[...]) + bl_ref[...] + _dot(x_ref[...], wr_ref[...])
    h = jnp.maximum(h, 0.0)
    outa_ref[...] = h[:, :128]
    outb_ref[...] = h[:, 128:]


def _layer1_body(agg_ref, degp_ref, h1a_ref, h1b_ref, wla_ref, wlb_ref,
                 bl_ref, wra_ref, wrb_ref, wl2_ref,
                 outa_ref, outb_ref, outm_ref):
    recip = _recip_deg(degp_ref)
    h = (_dot(agg_ref[0] * recip, wla_ref[...])
         + _dot(agg_ref[1] * recip, wlb_ref[...])
         + bl_ref[...]
         + _dot(h1a_ref[...], wra_ref[...])
         + _dot(h1b_ref[...], wrb_ref[...]))
    h = jnp.maximum(h, 0.0)
    outa_ref[...] = h[:, :128]
    outb_ref[...] = h[:, 128:]
    outm_ref[...] = _dot(h, wl2_ref[...])


def _layer2_body(pm_ref, degp_ref, h2a_ref, h2b_ref, wra_ref, wrb_ref,
                 bl_ref, out_ref):
    mean_wl = (pm_ref[0] + pm_ref[1]) * _recip_deg(degp_ref)
    out_ref[...] = (mean_wl + bl_ref[...]
                    + _dot(h2a_ref[...], wra_ref[...])
                    + _dot(h2b_ref[...], wrb_ref[...]))


def _node_spec(ch):
    return pl.BlockSpec((NC, BN, ch), lambda i: (0, i, 0))


def _row_spec(ch):
    return pl.BlockSpec((BN, ch), lambda i: (i, 0))


def _full_spec(shape):
    n = len(shape)
    return pl.BlockSpec(shape, lambda i: (0,) * n)


_GRID = (N_NODES // BN,)


def _layer0(pa, degp, x, wl, bl, wr):
    return pl.pallas_call(
        _layer0_body,
        grid=_GRID,
        in_specs=[_node_spec(128), _node_spec(128), _row_spec(128),
                  _full_spec(wl.shape), _full_spec(bl.shape),
                  _full_spec(wr.shape)],
        out_specs=[_row_spec(128), _row_spec(128)],
        out_shape=[jax.ShapeDtypeStruct((N_NODES, 128), jnp.float32)] * 2,
    )(pa, degp, x, wl, bl, wr)


def _layer1(agg, degp, h1a, h1b, wla, wlb, bl, wra, wrb, wl2):
    return pl.pallas_call(
        _layer1_body,
        grid=_GRID,
        in_specs=[_node_spec(128), _node_spec(128), _row_spec(128),
                  _row_spec(128), _full_spec(wla.shape),
                  _full_spec(wlb.shape), _full_spec(bl.shape),
                  _full_spec(wra.shape), _full_spec(wrb.shape),
                  _full_spec(wl2.shape)],
        out_specs=[_row_spec(128), _row_spec(128), _row_spec(128)],
        out_shape=[jax.ShapeDtypeStruct((N_NODES, 128), jnp.float32)] * 3,
    )(agg, degp, h1a, h1b, wla, wlb, bl, wra, wrb, wl2)


def _layer2(pm, degp, h2a, h2b, wra, wrb, bl):
    return pl.pallas_call(
        _layer2_body,
        grid=_GRID,
        in_specs=[_node_spec(128), _node_spec(128), _row_spec(128),
                  _row_spec(128), _full_spec(wra.shape),
                  _full_spec(wrb.shape), _full_spec(bl.shape)],
        out_specs=pl.BlockSpec((BN, 128), lambda i: (i, 0)),
        out_shape=jax.ShapeDtypeStruct((N_NODES, 128), jnp.float32),
    )(pm, degp, h2a, h2b, wra, wrb, bl)


@jax.jit
def kernel(x, adj_t, Wl0, bl0, Wr0, Wl1, bl1, Wr1, Wl2, bl2, Wr2):
    src = adj_t[0].astype(jnp.int32)
    dst = adj_t[1].astype(jnp.int32)
    npad = E_PAD - N_EDGES
    pad_ar = jnp.arange(npad, dtype=jnp.int32)
    src_p = jnp.concatenate([src, pad_ar % N_NODES])
    dst2 = jnp.concatenate(
        [dst, N_NODES + (pad_ar % (NP - N_NODES))]).reshape(-1, EB)
    z128 = jnp.zeros((RT, 128), jnp.float32)
    ones128 = jnp.ones((EB, 128), jnp.float32)

    degp = _deg_counts(dst2, z128, ones128)
    pa = _segsum_edges(x, src_p, dst2, z128)
    h1a, h1b = _layer0(pa, degp, x, Wl0, bl0.reshape(1, -1), Wr0)

    agg1 = _segsum_chans(h1a, h1b, src_p, dst2, z128)
    h2a, h2b, m = _layer1(agg1, degp, h1a, h1b, Wl1[:128], Wl1[128:],
                          bl1.reshape(1, -1), Wr1[:128], Wr1[128:], Wl2)

    pm = _segsum_edges(m, src_p, dst2, z128)
    out = _layer2(pm, degp, h2a, h2b, Wr2[:128], Wr2[128:],
                  bl2.reshape(1, -1))
    return out
